# baseline (speedup 1.0000x reference)
import jax
import jax.numpy as jnp
from jax import lax
from jax.experimental import pallas as pl
from jax.experimental.pallas import tpu as pltpu
from jax.experimental.pallas import tpu_sc as plsc

N_NODES = 10000
N_EDGES = 320000
D = 128
EPS_BN = 1e-5

NC = 2
NS = 16
NW = NC * NS
CH = 128
CPT = -(-N_EDGES // (NW * CH))
E_PAD = NW * CPT * CH
NROWS = 10112
RPT = NROWS // NS

_MESH = plsc.VectorSubcoreMesh(core_axis_name="c", subcore_axis_name="s")


def _sc_segment_sum():
    scratch = (
        pltpu.VMEM((CPT, CH), jnp.int32),
        pltpu.VMEM((CPT, CH), jnp.int32),
        pltpu.VMEM((CH, D), jnp.float32),
        pltpu.VMEM_SHARED((NROWS, D), jnp.float32),
        pltpu.SemaphoreType.DMA,
    )

    def body(feat, src_hbm, dst_hbm, zeros_hbm, agg_out,
             src_v, dst_v, gbuf, agg_sh, sem):
        c = lax.axis_index("c")
        s = lax.axis_index("s")
        wid = c * NS + s

        pltpu.sync_copy(src_hbm.at[wid], src_v)
        pltpu.sync_copy(dst_hbm.at[wid], dst_v)
        pltpu.sync_copy(zeros_hbm.at[pl.ds(s * RPT, RPT)],
                        agg_sh.at[pl.ds(s * RPT, RPT)])
        plsc.subcore_barrier()

        @pl.loop(0, CPT)
        def _chunk(j):
            pltpu.async_copy(feat.at[src_v.at[j]], gbuf, sem).wait()
            pltpu.sync_copy(gbuf, agg_sh.at[dst_v.at[j]], add=True)

        plsc.subcore_barrier()
        pltpu.sync_copy(agg_sh.at[pl.ds(s * RPT, RPT)],
                        agg_out.at[c, pl.ds(s * RPT, RPT)])

    return pl.kernel(
        body,
        out_type=jax.ShapeDtypeStruct((NC, NROWS, D), jnp.float32),
        mesh=_MESH, scratch_types=scratch)


def _tc_dense(bn: bool):
    def body(agg_ref, cnt_ref, x_ref, wl_ref, wr_ref, b_ref, g_ref, be_ref,
             out_ref):
        cnt = cnt_ref[0, :N_NODES, 0:1] + cnt_ref[1, :N_NODES, 0:1]
        inv = 1.0 / jnp.maximum(cnt, 1.0)
        agg = agg_ref[0, :N_NODES, :] + agg_ref[1, :N_NODES, :]
        mean = agg * inv
        t = (jnp.dot(mean, wl_ref[...], preferred_element_type=jnp.float32)
             + jnp.dot(x_ref[...], wr_ref[...], preferred_element_type=jnp.float32)
             + b_ref[...])
        if bn:
            m = jnp.mean(t, axis=0, keepdims=True)
            v = jnp.mean((t - m) * (t - m), axis=0, keepdims=True)
            t = (t - m) * lax.rsqrt(v + EPS_BN) * g_ref[...] + be_ref[...]
            t = jnp.maximum(t, 0.0)
        out_ref[...] = t

    return pl.pallas_call(
        body, out_shape=jax.ShapeDtypeStruct((N_NODES, D), jnp.float32))


_sc_sum = _sc_segment_sum()
_tc_bn = _tc_dense(True)
_tc_plain = _tc_dense(False)


def kernel(x, edge_index, Wl1, Wr1, b1, Wl2, Wr2, b2, Wl3, Wr3, b3,
           gamma1, beta1, gamma2, beta2):
    src = edge_index[0].astype(jnp.int32)
    dst = edge_index[1].astype(jnp.int32)
    pad = E_PAD - N_EDGES
    src_r = jnp.concatenate([src, jnp.zeros((pad,), jnp.int32)]).reshape(NW, CPT, CH)
    dst_r = jnp.concatenate([dst, jnp.full((pad,), N_NODES, jnp.int32)]).reshape(NW, CPT, CH)
    zeros = jnp.zeros((NROWS, D), jnp.float32)
    ones_tab = jnp.ones((8, D), jnp.float32)
    src0_r = jnp.zeros((NW, CPT, CH), jnp.int32)

    b1r, b2r, b3r = (b.reshape(1, D) for b in (b1, b2, b3))
    g1, g2 = gamma1.reshape(1, D), gamma2.reshape(1, D)
    be1, be2 = beta1.reshape(1, D), beta2.reshape(1, D)

    cnt = _sc_sum(ones_tab, src0_r, dst_r, zeros)
    agg1 = _sc_sum(x, src_r, dst_r, zeros)
    h1 = _tc_bn(agg1, cnt, x, Wl1, Wr1, b1r, g1, be1)
    agg2 = _sc_sum(h1, src_r, dst_r, zeros)
    h2 = _tc_bn(agg2, cnt, h1, Wl2, Wr2, b2r, g2, be2)
    agg3 = _sc_sum(h2, src_r, dst_r, zeros)
    out = _tc_plain(agg3, cnt, h2, Wl3, Wr3, b3r, g1, be1)
    return out


# speedup vs baseline: 11.2060x; 11.2060x over previous
import jax
import jax.numpy as jnp
from jax import lax
from jax.experimental import pallas as pl
from jax.experimental.pallas import tpu as pltpu
from jax.experimental.pallas import tpu_sc as plsc

N_NODES = 10000
N_EDGES = 320000
D = 128
EPS_BN = 1e-5

NC = 2
NS = 16
NW = NC * NS
CH = 128
CPT = -(-N_EDGES // (NW * CH))
E_PAD = NW * CPT * CH
NROWS = 10112
RPT = NROWS // NS

_MESH = plsc.VectorSubcoreMesh(core_axis_name="c", subcore_axis_name="s")


def _sc_segment_sum():
    scratch = (
        pltpu.VMEM((CPT, CH), jnp.int32),
        pltpu.VMEM((CPT, CH), jnp.int32),
        pltpu.VMEM((CH, D), jnp.float32),
        pltpu.VMEM_SHARED((NROWS, D), jnp.float32),
        pltpu.SemaphoreType.DMA,
    )

    def body(feat, src_hbm, dst_hbm, zeros_hbm, agg_out,
             src_v, dst_v, gbuf, agg_sh, sem):
        c = lax.axis_index("c")
        s = lax.axis_index("s")
        wid = c * NS + s

        pltpu.sync_copy(src_hbm.at[wid], src_v)
        pltpu.sync_copy(dst_hbm.at[wid], dst_v)
        pltpu.sync_copy(zeros_hbm.at[pl.ds(s * RPT, RPT)],
                        agg_sh.at[pl.ds(s * RPT, RPT)])
        plsc.subcore_barrier()

        @pl.loop(0, CPT)
        def _chunk(j):
            pltpu.async_copy(feat.at[src_v.at[j]], gbuf, sem).wait()
            pltpu.sync_copy(gbuf, agg_sh.at[dst_v.at[j]], add=True)

        plsc.subcore_barrier()
        pltpu.sync_copy(agg_sh.at[pl.ds(s * RPT, RPT)],
                        agg_out.at[c, pl.ds(s * RPT, RPT)])

    return pl.kernel(
        body,
        out_type=jax.ShapeDtypeStruct((NC, NROWS, D), jnp.float32),
        mesh=_MESH, scratch_types=scratch)


def _sc_count():
    scratch = (
        pltpu.VMEM((CPT, CH), jnp.int32),
        pltpu.VMEM((CH, D), jnp.float32),
        pltpu.VMEM_SHARED((NROWS, D), jnp.float32),
    )

    def body(dst_hbm, zeros_hbm, ones_hbm, cnt_out, dst_v, ones_v, cnt_sh):
        c = lax.axis_index("c")
        s = lax.axis_index("s")
        wid = c * NS + s

        pltpu.sync_copy(dst_hbm.at[wid], dst_v)
        pltpu.sync_copy(ones_hbm, ones_v)
        pltpu.sync_copy(zeros_hbm.at[pl.ds(s * RPT, RPT)],
                        cnt_sh.at[pl.ds(s * RPT, RPT)])
        plsc.subcore_barrier()

        @pl.loop(0, CPT)
        def _chunk(j):
            pltpu.sync_copy(ones_v, cnt_sh.at[dst_v.at[j]], add=True)

        plsc.subcore_barrier()
        pltpu.sync_copy(cnt_sh.at[pl.ds(s * RPT, RPT)],
                        cnt_out.at[c, pl.ds(s * RPT, RPT)])

    return pl.kernel(
        body,
        out_type=jax.ShapeDtypeStruct((NC, NROWS, D), jnp.float32),
        mesh=_MESH, scratch_types=scratch)


def _tc_dense(bn: bool):
    def body(agg_ref, cnt_ref, x_ref, wl_ref, wr_ref, b_ref, g_ref, be_ref,
             out_ref):
        cnt = cnt_ref[0, :N_NODES, 0:1] + cnt_ref[1, :N_NODES, 0:1]
        inv = 1.0 / jnp.maximum(cnt, 1.0)
        agg = agg_ref[0, :N_NODES, :] + agg_ref[1, :N_NODES, :]
        mean = agg * inv
        t = (jnp.dot(mean, wl_ref[...], preferred_element_type=jnp.float32)
             + jnp.dot(x_ref[...], wr_ref[...], preferred_element_type=jnp.float32)
             + b_ref[...])
        if bn:
            m = jnp.mean(t, axis=0, keepdims=True)
            v = jnp.mean((t - m) * (t - m), axis=0, keepdims=True)
            t = (t - m) * lax.rsqrt(v + EPS_BN) * g_ref[...] + be_ref[...]
            t = jnp.maximum(t, 0.0)
        out_ref[...] = t

    return pl.pallas_call(
        body, out_shape=jax.ShapeDtypeStruct((N_NODES, D), jnp.float32))


_sc_sum = _sc_segment_sum()
_sc_cnt = _sc_count()
_tc_bn = _tc_dense(True)
_tc_plain = _tc_dense(False)


def kernel(x, edge_index, Wl1, Wr1, b1, Wl2, Wr2, b2, Wl3, Wr3, b3,
           gamma1, beta1, gamma2, beta2):
    src = edge_index[0].astype(jnp.int32)
    dst = edge_index[1].astype(jnp.int32)
    pad = E_PAD - N_EDGES
    src_r = jnp.concatenate([src, jnp.zeros((pad,), jnp.int32)]).reshape(NW, CPT, CH)
    dst_r = jnp.concatenate([dst, jnp.full((pad,), N_NODES, jnp.int32)]).reshape(NW, CPT, CH)
    zeros = jnp.zeros((NROWS, D), jnp.float32)
    ones_tab = jnp.ones((CH, D), jnp.float32)

    b1r, b2r, b3r = (b.reshape(1, D) for b in (b1, b2, b3))
    g1, g2 = gamma1.reshape(1, D), gamma2.reshape(1, D)
    be1, be2 = beta1.reshape(1, D), beta2.reshape(1, D)

    cnt = _sc_cnt(dst_r, zeros, ones_tab)
    agg1 = _sc_sum(x, src_r, dst_r, zeros)
    h1 = _tc_bn(agg1, cnt, x, Wl1, Wr1, b1r, g1, be1)
    agg2 = _sc_sum(h1, src_r, dst_r, zeros)
    h2 = _tc_bn(agg2, cnt, h1, Wl2, Wr2, b2r, g2, be2)
    agg3 = _sc_sum(h2, src_r, dst_r, zeros)
    out = _tc_plain(agg3, cnt, h2, Wl3, Wr3, b3r, g1, be1)
    return out


# speedup vs baseline: 19.7029x; 1.7582x over previous
import jax
import jax.numpy as jnp
from jax import lax
from jax.experimental import pallas as pl
from jax.experimental.pallas import tpu as pltpu
from jax.experimental.pallas import tpu_sc as plsc

N_NODES = 10000
N_EDGES = 320000
D = 128
EPS_BN = 1e-5

NC = 2
NS = 16
NW = NC * NS
CH = 128
CPT = -(-N_EDGES // (NW * CH))
E_PAD = NW * CPT * CH
NROWS = 10112
RPT = NROWS // NS

_MESH = plsc.VectorSubcoreMesh(core_axis_name="c", subcore_axis_name="s")


def _sc_segment_sum():
    scratch = (
        pltpu.VMEM((CPT, CH), jnp.int32),
        pltpu.VMEM((CPT, CH), jnp.int32),
        pltpu.VMEM((CH, D), jnp.float32),
        pltpu.VMEM_SHARED((NROWS, D), jnp.float32),
        pltpu.SemaphoreType.DMA,
    )

    def body(feat, src_hbm, dst_hbm, zeros_hbm, agg_out,
             src_v, dst_v, gbuf, agg_sh, sem):
        c = lax.axis_index("c")
        s = lax.axis_index("s")
        wid = c * NS + s

        pltpu.sync_copy(src_hbm.at[wid], src_v)
        pltpu.sync_copy(dst_hbm.at[wid], dst_v)
        pltpu.sync_copy(zeros_hbm.at[pl.ds(s * RPT, RPT)],
                        agg_sh.at[pl.ds(s * RPT, RPT)])
        plsc.subcore_barrier()

        @pl.loop(0, CPT)
        def _chunk(j):
            pltpu.async_copy(feat.at[src_v.at[j]], gbuf, sem).wait()
            pltpu.sync_copy(gbuf, agg_sh.at[dst_v.at[j]], add=True)

        plsc.subcore_barrier()
        pltpu.sync_copy(agg_sh.at[pl.ds(s * RPT, RPT)],
                        agg_out.at[c, pl.ds(s * RPT, RPT)])

    return pl.kernel(
        body,
        out_type=jax.ShapeDtypeStruct((NC, NROWS, D), jnp.float32),
        mesh=_MESH, scratch_types=scratch)


def _sc_count():
    scratch = (
        pltpu.VMEM((CPT, CH), jnp.int32),
        pltpu.VMEM((CH, D), jnp.float32),
        pltpu.VMEM_SHARED((NROWS, D), jnp.float32),
    )

    def body(dst_hbm, zeros_hbm, ones_hbm, cnt_out, dst_v, ones_v, cnt_sh):
        c = lax.axis_index("c")
        s = lax.axis_index("s")
        wid = c * NS + s

        pltpu.sync_copy(dst_hbm.at[wid], dst_v)
        pltpu.sync_copy(ones_hbm, ones_v)
        pltpu.sync_copy(zeros_hbm.at[pl.ds(s * RPT, RPT)],
                        cnt_sh.at[pl.ds(s * RPT, RPT)])
        plsc.subcore_barrier()

        @pl.loop(0, CPT)
        def _chunk(j):
            pltpu.sync_copy(ones_v, cnt_sh.at[dst_v.at[j]], add=True)

        plsc.subcore_barrier()
        pltpu.sync_copy(cnt_sh.at[pl.ds(s * RPT, RPT)],
                        cnt_out.at[c, pl.ds(s * RPT, RPT)])

    return pl.kernel(
        body,
        out_type=jax.ShapeDtypeStruct((NC, NROWS, D), jnp.float32),
        mesh=_MESH, scratch_types=scratch)


def _tc_dense(bn: bool):
    def body(agg_ref, cnt_ref, x_ref, wl_ref, wr_ref, b_ref, g_ref, be_ref,
             out_ref):
        cnt = cnt_ref[0, :N_NODES, 0:1] + cnt_ref[1, :N_NODES, 0:1]
        inv = 1.0 / jnp.maximum(cnt, 1.0)
        agg = agg_ref[0, :N_NODES, :] + agg_ref[1, :N_NODES, :]
        mean = agg * inv
        t = (jnp.dot(mean, wl_ref[...], preferred_element_type=jnp.float32)
             + jnp.dot(x_ref[...], wr_ref[...], preferred_element_type=jnp.float32)
             + b_ref[...])
        if bn:
            m = jnp.mean(t, axis=0, keepdims=True)
            v = jnp.mean((t - m) * (t - m), axis=0, keepdims=True)
            t = (t - m) * lax.rsqrt(v + EPS_BN) * g_ref[...] + be_ref[...]
            t = jnp.maximum(t, 0.0)
        out_ref[...] = t

    return pl.pallas_call(
        body, out_shape=jax.ShapeDtypeStruct((N_NODES, D), jnp.float32))


_sc_sum = _sc_segment_sum()
_sc_cnt = _sc_count()
_tc_bn = _tc_dense(True)
_tc_plain = _tc_dense(False)


def kernel(x, edge_index, Wl1, Wr1, b1, Wl2, Wr2, b2, Wl3, Wr3, b3,
           gamma1, beta1, gamma2, beta2):
    src = edge_index[0].astype(jnp.int32)
    dst = edge_index[1].astype(jnp.int32)
    pad = E_PAD - N_EDGES
    pad_src = jnp.arange(pad, dtype=jnp.int32) % N_NODES
    pad_dst = N_NODES + jnp.arange(pad, dtype=jnp.int32) % (NROWS - N_NODES)
    src_r = jnp.concatenate([src, pad_src]).reshape(NW, CPT, CH)
    dst_r = jnp.concatenate([dst, pad_dst]).reshape(NW, CPT, CH)
    zeros = jnp.zeros((NROWS, D), jnp.float32)
    ones_tab = jnp.ones((CH, D), jnp.float32)

    b1r, b2r, b3r = (b.reshape(1, D) for b in (b1, b2, b3))
    g1, g2 = gamma1.reshape(1, D), gamma2.reshape(1, D)
    be1, be2 = beta1.reshape(1, D), beta2.reshape(1, D)

    cnt = _sc_cnt(dst_r, zeros, ones_tab)
    agg1 = _sc_sum(x, src_r, dst_r, zeros)
    h1 = _tc_bn(agg1, cnt, x, Wl1, Wr1, b1r, g1, be1)
    agg2 = _sc_sum(h1, src_r, dst_r, zeros)
    h2 = _tc_bn(agg2, cnt, h1, Wl2, Wr2, b2r, g2, be2)
    agg3 = _sc_sum(h2, src_r, dst_r, zeros)
    out = _tc_plain(agg3, cnt, h2, Wl3, Wr3, b3r, g1, be1)
    return out


# speedup vs baseline: 25.9116x; 1.3151x over previous
import jax
import jax.numpy as jnp
from jax import lax
from jax.experimental import pallas as pl
from jax.experimental.pallas import tpu as pltpu
from jax.experimental.pallas import tpu_sc as plsc

N_NODES = 10000
N_EDGES = 320000
D = 128
EPS_BN = 1e-5

NC = 2
NS = 16
NW = NC * NS
CH = 128
CPT = -(-N_EDGES // (NW * CH))
E_PAD = NW * CPT * CH
NROWS = 10112
RPT = NROWS // NS

_MESH = plsc.VectorSubcoreMesh(core_axis_name="c", subcore_axis_name="s")


def _sc_segment_sum():
    scratch = (
        pltpu.VMEM((2, 2, CH), jnp.int32),
        pltpu.VMEM((2, CH, D), jnp.float32),
        pltpu.VMEM_SHARED((NROWS, D), jnp.float32),
        pltpu.SemaphoreType.DMA,
        pltpu.SemaphoreType.DMA,
    )

    def body(feat, eidx_hbm, zeros_hbm, agg_out, ring, gbuf, agg_sh,
             gsem, isem):
        c = lax.axis_index("c")
        s = lax.axis_index("s")
        wid = c * NS + s

        pltpu.sync_copy(zeros_hbm.at[pl.ds(s * RPT, RPT)],
                        agg_sh.at[pl.ds(s * RPT, RPT)])
        plsc.subcore_barrier()

        pltpu.async_copy(eidx_hbm.at[wid, 0], ring.at[0], isem)
        pltpu.make_async_copy(eidx_hbm.at[wid, 0], ring.at[0], isem).wait()
        pltpu.async_copy(feat.at[ring.at[0, 0]], gbuf.at[0], gsem)
        pltpu.async_copy(eidx_hbm.at[wid, 1], ring.at[1], isem)

        @pl.loop(0, CPT - 1)
        def _chunk(j):
            b = lax.rem(j, 2)
            nb = lax.rem(j + 1, 2)
            pltpu.make_async_copy(eidx_hbm.at[wid, j + 1], ring.at[nb],
                                  isem).wait()
            pltpu.async_copy(feat.at[ring.at[nb, 0]], gbuf.at[nb], gsem)
            pltpu.make_async_copy(feat.at[ring.at[b, 0]], gbuf.at[b],
                                  gsem).wait()
            pltpu.sync_copy(gbuf.at[b], agg_sh.at[ring.at[b, 1]], add=True)
            pltpu.async_copy(eidx_hbm.at[wid, j + 2], ring.at[b], isem)

        bl = (CPT - 1) % 2
        pltpu.make_async_copy(feat.at[ring.at[bl, 0]], gbuf.at[bl],
                              gsem).wait()
        pltpu.sync_copy(gbuf.at[bl], agg_sh.at[ring.at[bl, 1]], add=True)
        pltpu.make_async_copy(eidx_hbm.at[wid, CPT], ring.at[bl], isem).wait()

        plsc.subcore_barrier()
        pltpu.sync_copy(agg_sh.at[pl.ds(s * RPT, RPT)],
                        agg_out.at[c, pl.ds(s * RPT, RPT)])

    return pl.kernel(
        body,
        out_type=jax.ShapeDtypeStruct((NC, NROWS, D), jnp.float32),
        mesh=_MESH, scratch_types=scratch)


def _sc_count():
    scratch = (
        pltpu.VMEM((CPT, CH), jnp.int32),
        pltpu.VMEM((CH, D), jnp.float32),
        pltpu.VMEM_SHARED((NROWS, D), jnp.float32),
    )

    def body(dst_hbm, zeros_hbm, ones_hbm, cnt_out, dst_v, ones_v, cnt_sh):
        c = lax.axis_index("c")
        s = lax.axis_index("s")
        wid = c * NS + s

        pltpu.sync_copy(dst_hbm.at[wid], dst_v)
        pltpu.sync_copy(ones_hbm, ones_v)
        pltpu.sync_copy(zeros_hbm.at[pl.ds(s * RPT, RPT)],
                        cnt_sh.at[pl.ds(s * RPT, RPT)])
        plsc.subcore_barrier()

        @pl.loop(0, CPT)
        def _chunk(j):
            pltpu.sync_copy(ones_v, cnt_sh.at[dst_v.at[j]], add=True)

        plsc.subcore_barrier()
        pltpu.sync_copy(cnt_sh.at[pl.ds(s * RPT, RPT)],
                        cnt_out.at[c, pl.ds(s * RPT, RPT)])

    return pl.kernel(
        body,
        out_type=jax.ShapeDtypeStruct((NC, NROWS, D), jnp.float32),
        mesh=_MESH, scratch_types=scratch)


def _tc_dense(bn: bool):
    def body(agg_ref, cnt_ref, x_ref, wl_ref, wr_ref, b_ref, g_ref, be_ref,
             out_ref):
        cnt = cnt_ref[0, :N_NODES, 0:1] + cnt_ref[1, :N_NODES, 0:1]
        inv = 1.0 / jnp.maximum(cnt, 1.0)
        agg = agg_ref[0, :N_NODES, :] + agg_ref[1, :N_NODES, :]
        mean = agg * inv
        t = (jnp.dot(mean, wl_ref[...], preferred_element_type=jnp.float32)
             + jnp.dot(x_ref[...], wr_ref[...], preferred_element_type=jnp.float32)
             + b_ref[...])
        if bn:
            m = jnp.mean(t, axis=0, keepdims=True)
            v = jnp.mean((t - m) * (t - m), axis=0, keepdims=True)
            t = (t - m) * lax.rsqrt(v + EPS_BN) * g_ref[...] + be_ref[...]
            t = jnp.maximum(t, 0.0)
        out_ref[...] = t

    return pl.pallas_call(
        body, out_shape=jax.ShapeDtypeStruct((N_NODES, D), jnp.float32))


_sc_sum = _sc_segment_sum()
_sc_cnt = _sc_count()
_tc_bn = _tc_dense(True)
_tc_plain = _tc_dense(False)


def kernel(x, edge_index, Wl1, Wr1, b1, Wl2, Wr2, b2, Wl3, Wr3, b3,
           gamma1, beta1, gamma2, beta2):
    src = edge_index[0].astype(jnp.int32)
    dst = edge_index[1].astype(jnp.int32)
    pad = E_PAD - N_EDGES
    pad_src = jnp.arange(pad, dtype=jnp.int32) % N_NODES
    pad_dst = N_NODES + jnp.arange(pad, dtype=jnp.int32) % (NROWS - N_NODES)
    src_r = jnp.concatenate([src, pad_src]).reshape(NW, CPT, CH)
    dst_r = jnp.concatenate([dst, pad_dst]).reshape(NW, CPT, CH)
    eidx = jnp.concatenate(
        [jnp.stack([src_r, dst_r], axis=2),
         jnp.zeros((NW, 1, 2, CH), jnp.int32)], axis=1)
    zeros = jnp.zeros((NROWS, D), jnp.float32)
    ones_tab = jnp.ones((CH, D), jnp.float32)

    b1r, b2r, b3r = (b.reshape(1, D) for b in (b1, b2, b3))
    g1, g2 = gamma1.reshape(1, D), gamma2.reshape(1, D)
    be1, be2 = beta1.reshape(1, D), beta2.reshape(1, D)

    cnt = _sc_cnt(dst_r, zeros, ones_tab)
    agg1 = _sc_sum(x, eidx, zeros)
    h1 = _tc_bn(agg1, cnt, x, Wl1, Wr1, b1r, g1, be1)
    agg2 = _sc_sum(h1, eidx, zeros)
    h2 = _tc_bn(agg2, cnt, h1, Wl2, Wr2, b2r, g2, be2)
    agg3 = _sc_sum(h2, eidx, zeros)
    out = _tc_plain(agg3, cnt, h2, Wl3, Wr3, b3r, g1, be1)
    return out


# speedup vs baseline: 28.3430x; 1.0938x over previous
import jax
import jax.numpy as jnp
from jax import lax
from jax.experimental import pallas as pl
from jax.experimental.pallas import tpu as pltpu
from jax.experimental.pallas import tpu_sc as plsc

N_NODES = 10000
N_EDGES = 320000
D = 128
EPS_BN = 1e-5

NC = 2
NS = 16
NW = NC * NS
CH = 128
CPT = -(-N_EDGES // (NW * CH))
E_PAD = NW * CPT * CH
NROWS = 10112
RPT = NROWS // NS

_MESH = plsc.VectorSubcoreMesh(core_axis_name="c", subcore_axis_name="s")


def _sc_segment_sum():
    scratch = (
        pltpu.VMEM((3, 2, CH), jnp.int32),
        pltpu.VMEM((2, CH, D), jnp.float32),
        pltpu.VMEM_SHARED((NROWS, D), jnp.float32),
        pltpu.SemaphoreType.DMA,
        pltpu.SemaphoreType.DMA,
        pltpu.SemaphoreType.DMA,
    )

    def body(feat, eidx_hbm, zeros_hbm, agg_out, ring, gbuf, agg_sh,
             gsem, isem, ssem):
        c = lax.axis_index("c")
        s = lax.axis_index("s")
        wid = c * NS + s

        def widx(j, slot):
            pltpu.make_async_copy(eidx_hbm.at[wid, j], ring.at[slot],
                                  isem).wait()

        def wgather(b):
            pltpu.make_async_copy(feat.at[ring.at[0, 0]], gbuf.at[b],
                                  gsem).wait()

        def wscatter(b):
            pltpu.make_async_copy(gbuf.at[b], agg_sh.at[ring.at[0, 1]],
                                  ssem).wait()

        pltpu.sync_copy(zeros_hbm.at[pl.ds(s * RPT, RPT)],
                        agg_sh.at[pl.ds(s * RPT, RPT)])
        plsc.subcore_barrier()

        pltpu.async_copy(eidx_hbm.at[wid, 0], ring.at[0], isem)
        widx(0, 0)
        pltpu.async_copy(feat.at[ring.at[0, 0]], gbuf.at[0], gsem)
        pltpu.async_copy(eidx_hbm.at[wid, 1], ring.at[1], isem)
        widx(1, 1)
        pltpu.async_copy(feat.at[ring.at[1, 0]], gbuf.at[1], gsem)
        wgather(0)
        pltpu.async_copy(gbuf.at[0], agg_sh.at[ring.at[0, 1]], ssem, add=True)
        pltpu.async_copy(eidx_hbm.at[wid, 2], ring.at[2], isem)

        @pl.loop(1, CPT - 1)
        def _chunk(j):
            b = lax.rem(j, 2)
            nb = 1 - b
            sj = lax.rem(j, 3)
            sj1 = lax.rem(j + 1, 3)
            sj2 = lax.rem(j + 2, 3)
            widx(j + 1, sj1)
            wscatter(nb)
            pltpu.async_copy(feat.at[ring.at[sj1, 0]], gbuf.at[nb], gsem)
            wgather(b)
            pltpu.async_copy(gbuf.at[b], agg_sh.at[ring.at[sj, 1]], ssem,
                             add=True)
            pltpu.async_copy(eidx_hbm.at[wid, j + 2], ring.at[sj2], isem)

        bl = (CPT - 1) % 2
        sl = (CPT - 1) % 3
        wscatter(1 - bl)
        wgather(bl)
        pltpu.async_copy(gbuf.at[bl], agg_sh.at[ring.at[sl, 1]], ssem,
                         add=True)
        wscatter(bl)
        widx(CPT, CPT % 3)

        plsc.subcore_barrier()
        pltpu.sync_copy(agg_sh.at[pl.ds(s * RPT, RPT)],
                        agg_out.at[c, pl.ds(s * RPT, RPT)])

    return pl.kernel(
        body,
        out_type=jax.ShapeDtypeStruct((NC, NROWS, D), jnp.float32),
        mesh=_MESH, scratch_types=scratch)


def _sc_count():
    scratch = (
        pltpu.VMEM((CPT, CH), jnp.int32),
        pltpu.VMEM((CH, D), jnp.float32),
        pltpu.VMEM_SHARED((NROWS, D), jnp.float32),
    )

    def body(dst_hbm, zeros_hbm, ones_hbm, cnt_out, dst_v, ones_v, cnt_sh):
        c = lax.axis_index("c")
        s = lax.axis_index("s")
        wid = c * NS + s

        pltpu.sync_copy(dst_hbm.at[wid], dst_v)
        pltpu.sync_copy(ones_hbm, ones_v)
        pltpu.sync_copy(zeros_hbm.at[pl.ds(s * RPT, RPT)],
                        cnt_sh.at[pl.ds(s * RPT, RPT)])
        plsc.subcore_barrier()

        @pl.loop(0, CPT)
        def _chunk(j):
            pltpu.sync_copy(ones_v, cnt_sh.at[dst_v.at[j]], add=True)

        plsc.subcore_barrier()
        pltpu.sync_copy(cnt_sh.at[pl.ds(s * RPT, RPT)],
                        cnt_out.at[c, pl.ds(s * RPT, RPT)])

    return pl.kernel(
        body,
        out_type=jax.ShapeDtypeStruct((NC, NROWS, D), jnp.float32),
        mesh=_MESH, scratch_types=scratch)


def _tc_dense(bn: bool):
    def body(agg_ref, cnt_ref, x_ref, wl_ref, wr_ref, b_ref, g_ref, be_ref,
             out_ref):
        cnt = cnt_ref[0, :N_NODES, 0:1] + cnt_ref[1, :N_NODES, 0:1]
        inv = 1.0 / jnp.maximum(cnt, 1.0)
        agg = agg_ref[0, :N_NODES, :] + agg_ref[1, :N_NODES, :]
        mean = agg * inv
        t = (jnp.dot(mean, wl_ref[...], preferred_element_type=jnp.float32)
             + jnp.dot(x_ref[...], wr_ref[...], preferred_element_type=jnp.float32)
             + b_ref[...])
        if bn:
            m = jnp.mean(t, axis=0, keepdims=True)
            v = jnp.mean((t - m) * (t - m), axis=0, keepdims=True)
            t = (t - m) * lax.rsqrt(v + EPS_BN) * g_ref[...] + be_ref[...]
            t = jnp.maximum(t, 0.0)
        out_ref[...] = t

    return pl.pallas_call(
        body, out_shape=jax.ShapeDtypeStruct((N_NODES, D), jnp.float32))


_sc_sum = _sc_segment_sum()
_sc_cnt = _sc_count()
_tc_bn = _tc_dense(True)
_tc_plain = _tc_dense(False)


def kernel(x, edge_index, Wl1, Wr1, b1, Wl2, Wr2, b2, Wl3, Wr3, b3,
           gamma1, beta1, gamma2, beta2):
    src = edge_index[0].astype(jnp.int32)
    dst = edge_index[1].astype(jnp.int32)
    pad = E_PAD - N_EDGES
    pad_src = jnp.arange(pad, dtype=jnp.int32) % N_NODES
    pad_dst = N_NODES + jnp.arange(pad, dtype=jnp.int32) % (NROWS - N_NODES)
    src_r = jnp.concatenate([src, pad_src]).reshape(NW, CPT, CH)
    dst_r = jnp.concatenate([dst, pad_dst]).reshape(NW, CPT, CH)
    eidx = jnp.concatenate(
        [jnp.stack([src_r, dst_r], axis=2),
         jnp.zeros((NW, 1, 2, CH), jnp.int32)], axis=1)
    zeros = jnp.zeros((NROWS, D), jnp.float32)
    ones_tab = jnp.ones((CH, D), jnp.float32)

    b1r, b2r, b3r = (b.reshape(1, D) for b in (b1, b2, b3))
    g1, g2 = gamma1.reshape(1, D), gamma2.reshape(1, D)
    be1, be2 = beta1.reshape(1, D), beta2.reshape(1, D)

    cnt = _sc_cnt(dst_r, zeros, ones_tab)
    agg1 = _sc_sum(x, eidx, zeros)
    h1 = _tc_bn(agg1, cnt, x, Wl1, Wr1, b1r, g1, be1)
    agg2 = _sc_sum(h1, eidx, zeros)
    h2 = _tc_bn(agg2, cnt, h1, Wl2, Wr2, b2r, g2, be2)
    agg3 = _sc_sum(h2, eidx, zeros)
    out = _tc_plain(agg3, cnt, h2, Wl3, Wr3, b3r, g1, be1)
    return out
